# baseline (speedup 1.0000x reference)
import functools

import jax
import jax.numpy as jnp
from jax import lax
from jax.experimental import pallas as pl
from jax.experimental.pallas import tpu as pltpu
from jax.experimental.pallas import tpu_sc as plsc

BATCH = 4096
HIST = 200
EMBED = 64
NUM_WORKERS = 32
SLICES = 4
SLICE_B = BATCH // SLICES
B_PER_W = SLICE_B // NUM_WORKERS
GCHUNK = 40
N_GCHUNK = HIST // GCHUNK
NBUF = 4


def _sc_gather(table, idx):
    mesh = plsc.VectorSubcoreMesh(core_axis_name="c", subcore_axis_name="s")

    @functools.partial(
        pl.kernel,
        mesh=mesh,
        out_type=jax.ShapeDtypeStruct((SLICE_B, HIST, EMBED), jnp.float32),
        scratch_types=[
            pltpu.VMEM((B_PER_W, HIST), jnp.int32),
            pltpu.VMEM((NBUF, HIST, EMBED), jnp.float32),
            pltpu.SemaphoreType.DMA((NBUF,)),
            pltpu.SemaphoreType.DMA((NBUF,)),
        ],
        compiler_params=pltpu.CompilerParams(use_tc_tiling_on_sc=False),
    )
    def k(table_hbm, idx_hbm, out_hbm, idx_v, rows_v, gsem, ssem):
        wid = lax.axis_index("s") * 2 + lax.axis_index("c")
        b0 = wid * B_PER_W

        pltpu.sync_copy(idx_hbm.at[pl.ds(b0, B_PER_W)], idx_v)

        def start_gathers(b_local, k_buf):
            for c in range(N_GCHUNK):
                pltpu.async_copy(
                    table_hbm.at[idx_v.at[b_local, pl.ds(c * GCHUNK, GCHUNK)]],
                    rows_v.at[k_buf, pl.ds(c * GCHUNK, GCHUNK)],
                    gsem.at[k_buf],
                )

        for k_buf in range(NBUF):
            start_gathers(k_buf, k_buf)

        def outer(o, carry):
            for k_buf in range(NBUF):
                b = o * NBUF + k_buf
                dst = out_hbm.at[b0 + b]
                pltpu.make_async_copy(dst, rows_v.at[k_buf], gsem.at[k_buf]).wait()
                pltpu.async_copy(rows_v.at[k_buf], dst, ssem.at[k_buf])
                j = b + NBUF

                @pl.when(j < B_PER_W)
                def _():
                    pltpu.make_async_copy(
                        rows_v.at[k_buf], dst, ssem.at[k_buf]).wait()
                    start_gathers(j, k_buf)

            return carry

        lax.fori_loop(0, B_PER_W // NBUF, outer, 0)

        for k_buf in range(NBUF):
            pltpu.make_async_copy(
                rows_v.at[k_buf], out_hbm.at[b0], ssem.at[k_buf]).wait()

    return k(table, idx)


def kernel(inputs, embeddings):
    idx = inputs.astype(jnp.int32)
    parts = [
        _sc_gather(embeddings, idx[s * SLICE_B:(s + 1) * SLICE_B])
        for s in range(SLICES)
    ]
    return jnp.concatenate(parts, axis=0)


# speedup vs baseline: 1.2393x; 1.2393x over previous
import functools

import jax
import jax.numpy as jnp
from jax import lax
from jax.experimental import pallas as pl
from jax.experimental.pallas import tpu as pltpu
from jax.experimental.pallas import tpu_sc as plsc

BATCH = 4096
HIST = 200
EMBED = 64
NUM_WORKERS = 32
B_PER_W = BATCH // NUM_WORKERS
GCHUNK = 40
N_GCHUNK = HIST // GCHUNK
NBUF = 4


def _sc_gather(table, idx):
    mesh = plsc.VectorSubcoreMesh(core_axis_name="c", subcore_axis_name="s")

    @functools.partial(
        pl.kernel,
        mesh=mesh,
        out_type=jax.ShapeDtypeStruct((BATCH, HIST, EMBED), jnp.float32),
        scratch_types=[
            pltpu.VMEM((B_PER_W, HIST), jnp.int32),
            pltpu.VMEM((NBUF, HIST, EMBED), jnp.float32),
            pltpu.SemaphoreType.DMA((NBUF,)),
            pltpu.SemaphoreType.DMA((NBUF,)),
        ],
        compiler_params=pltpu.CompilerParams(use_tc_tiling_on_sc=False),
    )
    def k(table_hbm, idx_hbm, out_hbm, idx_v, rows_v, gsem, ssem):
        wid = lax.axis_index("s") * 2 + lax.axis_index("c")
        b0 = wid * B_PER_W

        pltpu.sync_copy(idx_hbm.at[pl.ds(b0, B_PER_W)], idx_v)

        def start_gathers(b_local, k_buf):
            for c in range(N_GCHUNK):
                pltpu.async_copy(
                    table_hbm.at[idx_v.at[b_local, pl.ds(c * GCHUNK, GCHUNK)]],
                    rows_v.at[k_buf, pl.ds(c * GCHUNK, GCHUNK)],
                    gsem.at[k_buf],
                )

        for k_buf in range(NBUF):
            start_gathers(k_buf, k_buf)

        def outer(o, carry):
            for k_buf in range(NBUF):
                b = o * NBUF + k_buf
                dst = out_hbm.at[b0 + b]
                pltpu.make_async_copy(dst, rows_v.at[k_buf], gsem.at[k_buf]).wait()
                pltpu.async_copy(rows_v.at[k_buf], dst, ssem.at[k_buf])
                j = b + NBUF

                @pl.when(j < B_PER_W)
                def _():
                    pltpu.make_async_copy(
                        rows_v.at[k_buf], dst, ssem.at[k_buf]).wait()
                    start_gathers(j, k_buf)

            return carry

        lax.fori_loop(0, B_PER_W // NBUF, outer, 0)

        for k_buf in range(NBUF):
            pltpu.make_async_copy(
                rows_v.at[k_buf], out_hbm.at[b0], ssem.at[k_buf]).wait()

    return k(table, idx)


def kernel(inputs, embeddings):
    return _sc_gather(embeddings, inputs.astype(jnp.int32))
